# baseline (speedup 1.0000x reference)
import functools

import jax
import jax.numpy as jnp
from jax.experimental import pallas as pl
from jax.experimental.pallas import tpu as pltpu

_TOP_K = 8


def _bdot(a, b):
    return jax.lax.dot_general(
        a.astype(jnp.bfloat16), b.astype(jnp.bfloat16),
        (((1,), (0,)), ((), ())),
        preferred_element_type=jnp.float32)


def _fused_body(x_ref, xb_ref, sfp_ref, tfp_ref,
                w1x_ref, w1p_ref, bf1_ref, wf2_ref, bf2_ref,
                wcs_ref, wct_ref, bc_ref,
                wg1a_ref, wg1b_ref, wg1c_ref, bg1_ref,
                wg2_ref, bg2_ref, wr_ref, br_ref,
                out_ref):
    x = x_ref[...]
    sfp = sfp_ref[...]
    tfp = tfp_ref[...]

    hx = _bdot(xb_ref[...], w1x_ref[...]) + bf1_ref[...]
    hs = jax.nn.gelu(hx + _bdot(sfp, w1p_ref[...]))
    ht = jax.nn.gelu(hx + _bdot(tfp, w1p_ref[...]))
    src_pot = _bdot(hs, wf2_ref[...]) + bf2_ref[...]
    tgt_pot = _bdot(ht, wf2_ref[...]) + bf2_ref[...]

    compat = _bdot(sfp, wcs_ref[...]) + _bdot(tfp, wct_ref[...]) + bc_ref[...]
    mod_tgt = tgt_pot * jax.nn.sigmoid(compat)

    g1 = jax.nn.gelu(_bdot(src_pot, wg1a_ref[...]) + _bdot(mod_tgt, wg1b_ref[...])
                     + _bdot(sfp, wg1c_ref[...]) + bg1_ref[...])
    gate = jax.nn.sigmoid(_bdot(g1, wg2_ref[...]) + bg2_ref[...])

    logits = _bdot(x * gate, wr_ref[...]) + br_ref[...]

    cur = logits
    neg = jnp.float32(-jnp.inf)
    thresh = None
    for _ in range(_TOP_K):
        thresh = jnp.max(cur, axis=-1, keepdims=True)
        cur = jnp.where(cur >= thresh, neg, cur)
    rowmax = jnp.max(logits, axis=-1, keepdims=True)
    e = jnp.where(logits >= thresh, jnp.exp(logits - rowmax), 0.0)
    out_ref[...] = e / jnp.sum(e, axis=-1, keepdims=True)


@functools.partial(jax.jit, static_argnames=())
def kernel(x, source_fingerprint, target_fingerprint,
           Wf1, bf1, Wf2, bf2, Wc, bc, Wg1, bg1, Wg2, bg2, Wr, br):
    T, FEAT = x.shape
    FP = source_fingerprint.shape[1]
    H = Wf1.shape[1]
    NF = Wf2.shape[1]
    E = Wr.shape[1]

    bf16 = jnp.bfloat16
    w1x, w1p = Wf1[:FEAT].astype(bf16), Wf1[FEAT:].astype(bf16)
    wcs, wct = Wc[:FP].astype(bf16), Wc[FP:].astype(bf16)
    wg1a = Wg1[:NF].astype(bf16)
    wg1b = Wg1[NF:2 * NF].astype(bf16)
    wg1c = Wg1[2 * NF:].astype(bf16)
    wf2b = Wf2.astype(bf16)
    wg2b = Wg2.astype(bf16)
    wrb = Wr.astype(bf16)
    xb = x.astype(bf16)
    sfpb = source_fingerprint.astype(bf16)
    tfpb = target_fingerprint.astype(bf16)
    bf1r = bf1.reshape(1, H)
    bf2r = bf2.reshape(1, NF)
    bcr = bc.reshape(1, NF)
    bg1r = bg1.reshape(1, H)
    bg2r = bg2.reshape(1, FEAT)
    brr = br.reshape(1, E)

    TB = 256
    grid = (T // TB,)

    def tok_spec(cols):
        return pl.BlockSpec((TB, cols), lambda i: (i, 0))

    def full_spec(arr):
        return pl.BlockSpec(arr.shape, lambda i: (0,) * arr.ndim)

    weights = (w1x, w1p, bf1r, wf2b, bf2r, wcs, wct, bcr,
               wg1a, wg1b, wg1c, bg1r, wg2b, bg2r, wrb, brr)

    return pl.pallas_call(
        _fused_body,
        grid=grid,
        in_specs=[tok_spec(FEAT), tok_spec(FEAT), tok_spec(FP), tok_spec(FP)]
                 + [full_spec(w) for w in weights],
        out_specs=tok_spec(E),
        out_shape=jax.ShapeDtypeStruct((T, E), jnp.float32),
        compiler_params=pltpu.CompilerParams(
            dimension_semantics=("arbitrary",),
        ),
    )(x, xb, sfpb, tfpb, *weights)


# speedup vs baseline: 1.2202x; 1.2202x over previous
import functools

import jax
import jax.numpy as jnp
from jax.experimental import pallas as pl
from jax.experimental.pallas import tpu as pltpu

_TOP_K = 8


def _bdot(a, b):
    return jax.lax.dot_general(
        a.astype(jnp.bfloat16), b.astype(jnp.bfloat16),
        (((1,), (0,)), ((), ())),
        preferred_element_type=jnp.float32)


def _fused_body(x_ref, sfp_ref, tfp_ref,
                w1x_ref, w1p_ref, bf1_ref, wf2_ref, bf2_ref,
                wcs_ref, wct_ref, bc_ref,
                wg1a_ref, wg1b_ref, wg1c_ref, bg1_ref,
                wg2_ref, bg2_ref, wr_ref, br_ref,
                out_ref):
    x = x_ref[...]
    sfp = sfp_ref[...]
    tfp = tfp_ref[...]

    hx = _bdot(x, w1x_ref[...]) + bf1_ref[...]
    hs = jax.nn.gelu(hx + _bdot(sfp, w1p_ref[...]))
    ht = jax.nn.gelu(hx + _bdot(tfp, w1p_ref[...]))
    src_pot = _bdot(hs, wf2_ref[...]) + bf2_ref[...]
    tgt_pot = _bdot(ht, wf2_ref[...]) + bf2_ref[...]

    compat = _bdot(sfp, wcs_ref[...]) + _bdot(tfp, wct_ref[...]) + bc_ref[...]
    mod_tgt = tgt_pot * jax.nn.sigmoid(compat)

    g1 = jax.nn.gelu(_bdot(src_pot, wg1a_ref[...]) + _bdot(mod_tgt, wg1b_ref[...])
                     + _bdot(sfp, wg1c_ref[...]) + bg1_ref[...])
    gate = jax.nn.sigmoid(_bdot(g1, wg2_ref[...]) + bg2_ref[...])

    logits = _bdot(x * gate, wr_ref[...]) + br_ref[...]

    cur = logits
    neg = jnp.float32(-jnp.inf)
    thresh = None
    for _ in range(_TOP_K):
        thresh = jnp.max(cur, axis=-1, keepdims=True)
        cur = jnp.where(cur >= thresh, neg, cur)
    rowmax = jnp.max(logits, axis=-1, keepdims=True)
    e = jnp.where(logits >= thresh, jnp.exp(logits - rowmax), 0.0)
    out_ref[...] = e / jnp.sum(e, axis=-1, keepdims=True)


@functools.partial(jax.jit, static_argnames=())
def kernel(x, source_fingerprint, target_fingerprint,
           Wf1, bf1, Wf2, bf2, Wc, bc, Wg1, bg1, Wg2, bg2, Wr, br):
    T, FEAT = x.shape
    FP = source_fingerprint.shape[1]
    H = Wf1.shape[1]
    NF = Wf2.shape[1]
    E = Wr.shape[1]

    w1x, w1p = Wf1[:FEAT], Wf1[FEAT:]
    wcs, wct = Wc[:FP], Wc[FP:]
    wg1a, wg1b, wg1c = Wg1[:NF], Wg1[NF:2 * NF], Wg1[2 * NF:]
    bf1r = bf1.reshape(1, H)
    bf2r = bf2.reshape(1, NF)
    bcr = bc.reshape(1, NF)
    bg1r = bg1.reshape(1, H)
    bg2r = bg2.reshape(1, FEAT)
    brr = br.reshape(1, E)

    TB = 256
    grid = (T // TB,)

    def tok_spec(cols):
        return pl.BlockSpec((TB, cols), lambda i: (i, 0))

    def full_spec(arr):
        return pl.BlockSpec(arr.shape, lambda i: (0,) * arr.ndim)

    weights = (w1x, w1p, bf1r, Wf2, bf2r, wcs, wct, bcr,
               wg1a, wg1b, wg1c, bg1r, Wg2, bg2r, Wr, brr)

    return pl.pallas_call(
        _fused_body,
        grid=grid,
        in_specs=[tok_spec(FEAT), tok_spec(FP), tok_spec(FP)]
                 + [full_spec(w) for w in weights],
        out_specs=tok_spec(E),
        out_shape=jax.ShapeDtypeStruct((T, E), jnp.float32),
        compiler_params=pltpu.CompilerParams(
            dimension_semantics=("parallel",),
        ),
    )(x, source_fingerprint, target_fingerprint, *weights)


# speedup vs baseline: 1.2244x; 1.0034x over previous
import functools

import jax
import jax.numpy as jnp
from jax.experimental import pallas as pl
from jax.experimental.pallas import tpu as pltpu

_TOP_K = 8


def _bdot(a, b):
    return jax.lax.dot_general(
        a.astype(jnp.bfloat16), b.astype(jnp.bfloat16),
        (((1,), (0,)), ((), ())),
        preferred_element_type=jnp.float32)


def _fused_body(x_ref, sfp_ref, tfp_ref,
                w1x_ref, w1p_ref, bf1_ref, wf2_ref, bf2_ref,
                wcs_ref, wct_ref, bc_ref,
                wg1a_ref, wg1b_ref, wg1c_ref, bg1_ref,
                wg2_ref, bg2_ref, wr_ref, br_ref,
                out_ref,
                w1xb_s, wg2b_s, wrb_s):
    @pl.when(pl.program_id(0) == 0)
    def _cast_weights():
        w1xb_s[...] = w1x_ref[...].astype(jnp.bfloat16)
        wg2b_s[...] = wg2_ref[...].astype(jnp.bfloat16)
        wrb_s[...] = wr_ref[...].astype(jnp.bfloat16)

    x = x_ref[...]
    sfp = sfp_ref[...]
    tfp = tfp_ref[...]

    hx = _bdot(x, w1xb_s[...]) + bf1_ref[...]
    hs = jax.nn.gelu(hx + _bdot(sfp, w1p_ref[...]))
    ht = jax.nn.gelu(hx + _bdot(tfp, w1p_ref[...]))
    src_pot = _bdot(hs, wf2_ref[...]) + bf2_ref[...]
    tgt_pot = _bdot(ht, wf2_ref[...]) + bf2_ref[...]

    compat = _bdot(sfp, wcs_ref[...]) + _bdot(tfp, wct_ref[...]) + bc_ref[...]
    mod_tgt = tgt_pot * jax.nn.sigmoid(compat)

    g1 = jax.nn.gelu(_bdot(src_pot, wg1a_ref[...]) + _bdot(mod_tgt, wg1b_ref[...])
                     + _bdot(sfp, wg1c_ref[...]) + bg1_ref[...])
    gate = jax.nn.sigmoid(_bdot(g1, wg2b_s[...]) + bg2_ref[...])

    logits = _bdot(x * gate, wrb_s[...]) + br_ref[...]

    cur = logits
    neg = jnp.float32(-jnp.inf)
    thresh = None
    for _ in range(_TOP_K):
        thresh = jnp.max(cur, axis=-1, keepdims=True)
        cur = jnp.where(cur >= thresh, neg, cur)
    rowmax = jnp.max(logits, axis=-1, keepdims=True)
    e = jnp.where(logits >= thresh, jnp.exp(logits - rowmax), 0.0)
    out_ref[...] = e / jnp.sum(e, axis=-1, keepdims=True)


@functools.partial(jax.jit, static_argnames=())
def kernel(x, source_fingerprint, target_fingerprint,
           Wf1, bf1, Wf2, bf2, Wc, bc, Wg1, bg1, Wg2, bg2, Wr, br):
    T, FEAT = x.shape
    FP = source_fingerprint.shape[1]
    H = Wf1.shape[1]
    NF = Wf2.shape[1]
    E = Wr.shape[1]

    w1x, w1p = Wf1[:FEAT], Wf1[FEAT:]
    wcs, wct = Wc[:FP], Wc[FP:]
    wg1a, wg1b, wg1c = Wg1[:NF], Wg1[NF:2 * NF], Wg1[2 * NF:]
    bf1r = bf1.reshape(1, H)
    bf2r = bf2.reshape(1, NF)
    bcr = bc.reshape(1, NF)
    bg1r = bg1.reshape(1, H)
    bg2r = bg2.reshape(1, FEAT)
    brr = br.reshape(1, E)

    TB = 256
    grid = (T // TB,)

    def tok_spec(cols):
        return pl.BlockSpec((TB, cols), lambda i: (i, 0))

    def full_spec(arr):
        return pl.BlockSpec(arr.shape, lambda i: (0,) * arr.ndim)

    weights = (w1x, w1p, bf1r, Wf2, bf2r, wcs, wct, bcr,
               wg1a, wg1b, wg1c, bg1r, Wg2, bg2r, Wr, brr)

    return pl.pallas_call(
        _fused_body,
        grid=grid,
        in_specs=[tok_spec(FEAT), tok_spec(FP), tok_spec(FP)]
                 + [full_spec(w) for w in weights],
        out_specs=tok_spec(E),
        out_shape=jax.ShapeDtypeStruct((T, E), jnp.float32),
        scratch_shapes=[
            pltpu.VMEM((FEAT, H), jnp.bfloat16),
            pltpu.VMEM((H, FEAT), jnp.bfloat16),
            pltpu.VMEM((FEAT, E), jnp.bfloat16),
        ],
        compiler_params=pltpu.CompilerParams(
            dimension_semantics=("arbitrary",),
        ),
    )(x, source_fingerprint, target_fingerprint, *weights)


# speedup vs baseline: 1.3928x; 1.1376x over previous
import functools

import jax
import jax.numpy as jnp
from jax.experimental import pallas as pl
from jax.experimental.pallas import tpu as pltpu

_TOP_K = 8


def _bdot(a, b):
    return jax.lax.dot_general(
        a.astype(jnp.bfloat16), b.astype(jnp.bfloat16),
        (((1,), (0,)), ((), ())),
        preferred_element_type=jnp.float32)


def _fused_body(x_ref, sfp_ref, tfp_ref,
                w1x_ref, w1p_ref, bf1_ref, wf2_ref, bf2_ref,
                wcs_ref, wct_ref, bc_ref,
                wg1a_ref, wg1b_ref, wg1c_ref, bg1_ref,
                wg2_ref, bg2_ref, wr_ref, br_ref,
                out_ref,
                w1xb_s, wg2b_s, wrb_s):
    @pl.when(pl.program_id(0) == 0)
    def _cast_weights():
        w1xb_s[...] = w1x_ref[...].astype(jnp.bfloat16)
        wg2b_s[...] = wg2_ref[...].astype(jnp.bfloat16)
        wrb_s[...] = wr_ref[...].astype(jnp.bfloat16)

    x = x_ref[...]
    sfp = sfp_ref[...]
    tfp = tfp_ref[...]

    hx = _bdot(x, w1xb_s[...]) + bf1_ref[...]
    hs = jax.nn.gelu(hx + _bdot(sfp, w1p_ref[...]))
    ht = jax.nn.gelu(hx + _bdot(tfp, w1p_ref[...]))
    src_pot = _bdot(hs, wf2_ref[...]) + bf2_ref[...]
    tgt_pot = _bdot(ht, wf2_ref[...]) + bf2_ref[...]

    compat = _bdot(sfp, wcs_ref[...]) + _bdot(tfp, wct_ref[...]) + bc_ref[...]
    mod_tgt = tgt_pot * jax.nn.sigmoid(compat)

    g1 = jax.nn.gelu(_bdot(src_pot, wg1a_ref[...]) + _bdot(mod_tgt, wg1b_ref[...])
                     + _bdot(sfp, wg1c_ref[...]) + bg1_ref[...])
    gate = jax.nn.sigmoid(_bdot(g1, wg2b_s[...]) + bg2_ref[...])

    logits = _bdot(x * gate, wrb_s[...]) + br_ref[...]

    cur = logits
    neg = jnp.float32(-jnp.inf)
    thresh = None
    for _ in range(_TOP_K):
        thresh = jnp.max(cur, axis=-1, keepdims=True)
        cur = jnp.where(cur >= thresh, neg, cur)
    rowmax = jnp.max(logits, axis=-1, keepdims=True)
    e = jnp.where(logits >= thresh, jnp.exp(logits - rowmax), 0.0)
    out_ref[...] = e / jnp.sum(e, axis=-1, keepdims=True)


@functools.partial(jax.jit, static_argnames=())
def kernel(x, source_fingerprint, target_fingerprint,
           Wf1, bf1, Wf2, bf2, Wc, bc, Wg1, bg1, Wg2, bg2, Wr, br):
    T, FEAT = x.shape
    FP = source_fingerprint.shape[1]
    H = Wf1.shape[1]
    NF = Wf2.shape[1]
    E = Wr.shape[1]

    w1x, w1p = Wf1[:FEAT], Wf1[FEAT:]
    wcs, wct = Wc[:FP], Wc[FP:]
    wg1a, wg1b, wg1c = Wg1[:NF], Wg1[NF:2 * NF], Wg1[2 * NF:]
    bf1r = bf1.reshape(1, H)
    bf2r = bf2.reshape(1, NF)
    bcr = bc.reshape(1, NF)
    bg1r = bg1.reshape(1, H)
    bg2r = bg2.reshape(1, FEAT)
    brr = br.reshape(1, E)

    TB = 512
    grid = (T // TB,)

    def tok_spec(cols):
        return pl.BlockSpec((TB, cols), lambda i: (i, 0))

    def full_spec(arr):
        return pl.BlockSpec(arr.shape, lambda i: (0,) * arr.ndim)

    weights = (w1x, w1p, bf1r, Wf2, bf2r, wcs, wct, bcr,
               wg1a, wg1b, wg1c, bg1r, Wg2, bg2r, Wr, brr)

    return pl.pallas_call(
        _fused_body,
        grid=grid,
        in_specs=[tok_spec(FEAT), tok_spec(FP), tok_spec(FP)]
                 + [full_spec(w) for w in weights],
        out_specs=tok_spec(E),
        out_shape=jax.ShapeDtypeStruct((T, E), jnp.float32),
        scratch_shapes=[
            pltpu.VMEM((FEAT, H), jnp.bfloat16),
            pltpu.VMEM((H, FEAT), jnp.bfloat16),
            pltpu.VMEM((FEAT, E), jnp.bfloat16),
        ],
        compiler_params=pltpu.CompilerParams(
            dimension_semantics=("arbitrary",),
        ),
    )(x, source_fingerprint, target_fingerprint, *weights)


# speedup vs baseline: 1.5013x; 1.0779x over previous
import functools

import jax
import jax.numpy as jnp
from jax.experimental import pallas as pl
from jax.experimental.pallas import tpu as pltpu

_TOP_K = 8
_HALF = 256


def _bdot(a, b):
    return jax.lax.dot_general(
        a.astype(jnp.bfloat16), b.astype(jnp.bfloat16),
        (((1,), (0,)), ((), ())),
        preferred_element_type=jnp.float32)


def _fused_body(x_ref, sfp_ref, tfp_ref,
                w1x_ref, w1p_ref, bf1_ref, wf2_ref, bf2_ref,
                wcs_ref, wct_ref, bc_ref,
                wg1a_ref, wg1b_ref, wg1c_ref, bg1_ref,
                wg2_ref, bg2_ref, wr_ref, br_ref,
                out_ref,
                w1xb_s, wg2b_s, wrb_s):
    @pl.when(pl.program_id(0) == 0)
    def _cast_weights():
        w1xb_s[...] = w1x_ref[...].astype(jnp.bfloat16)
        wg2b_s[...] = wg2_ref[...].astype(jnp.bfloat16)
        wrb_s[...] = wr_ref[...].astype(jnp.bfloat16)

    x = x_ref[...]
    sfp = sfp_ref[...]
    tfp = tfp_ref[...]

    hx = _bdot(x, w1xb_s[...]) + bf1_ref[...]
    hs = jax.nn.gelu(hx + _bdot(sfp, w1p_ref[...]))
    ht = jax.nn.gelu(hx + _bdot(tfp, w1p_ref[...]))
    src_pot = _bdot(hs, wf2_ref[...]) + bf2_ref[...]
    tgt_pot = _bdot(ht, wf2_ref[...]) + bf2_ref[...]

    compat = _bdot(sfp, wcs_ref[...]) + _bdot(tfp, wct_ref[...]) + bc_ref[...]
    mod_tgt = tgt_pot * jax.nn.sigmoid(compat)

    g1 = jax.nn.gelu(_bdot(src_pot, wg1a_ref[...]) + _bdot(mod_tgt, wg1b_ref[...])
                     + _bdot(sfp, wg1c_ref[...]) + bg1_ref[...])
    zg = _bdot(g1, wg2b_s[...]) + bg2_ref[...]
    gate = 0.5 + 0.5 * jnp.tanh(0.5 * zg)

    logits = _bdot(x * gate, wrb_s[...]) + br_ref[...]

    cur = logits
    neg = jnp.float32(-jnp.inf)
    thresh = None
    for _ in range(_TOP_K):
        thresh = jnp.max(cur, axis=-1, keepdims=True)
        cur = jnp.where(cur >= thresh, neg, cur)
    rowmax = jnp.max(logits, axis=-1, keepdims=True)
    e = jnp.where(logits >= thresh, jnp.exp(logits - rowmax), 0.0)
    out_ref[...] = e / jnp.sum(e, axis=-1, keepdims=True)


@functools.partial(jax.jit, static_argnames=())
def kernel(x, source_fingerprint, target_fingerprint,
           Wf1, bf1, Wf2, bf2, Wc, bc, Wg1, bg1, Wg2, bg2, Wr, br):
    T, FEAT = x.shape
    FP = source_fingerprint.shape[1]
    H = Wf1.shape[1]
    NF = Wf2.shape[1]
    E = Wr.shape[1]

    w1x, w1p = Wf1[:FEAT], Wf1[FEAT:]
    wcs, wct = Wc[:FP], Wc[FP:]
    wg1a, wg1b, wg1c = Wg1[:NF], Wg1[NF:2 * NF], Wg1[2 * NF:]
    bf1r = bf1.reshape(1, H)
    bf2r = bf2.reshape(1, NF)
    bcr = bc.reshape(1, NF)
    bg1r = bg1.reshape(1, H)
    bg2r = bg2.reshape(1, FEAT)
    brr = br.reshape(1, E)

    TB = 1024
    grid = (T // TB,)

    def tok_spec(cols):
        return pl.BlockSpec((TB, cols), lambda i: (i, 0))

    def full_spec(arr):
        return pl.BlockSpec(arr.shape, lambda i: (0,) * arr.ndim)

    weights = (w1x, w1p, bf1r, Wf2, bf2r, wcs, wct, bcr,
               wg1a, wg1b, wg1c, bg1r, Wg2, bg2r, Wr, brr)

    return pl.pallas_call(
        _fused_body,
        grid=grid,
        in_specs=[tok_spec(FEAT), tok_spec(FP), tok_spec(FP)]
                 + [full_spec(w) for w in weights],
        out_specs=tok_spec(E),
        out_shape=jax.ShapeDtypeStruct((T, E), jnp.float32),
        scratch_shapes=[
            pltpu.VMEM((FEAT, H), jnp.bfloat16),
            pltpu.VMEM((H, FEAT), jnp.bfloat16),
            pltpu.VMEM((FEAT, E), jnp.bfloat16),
        ],
        compiler_params=pltpu.CompilerParams(
            dimension_semantics=("arbitrary",),
        ),
    )(x, source_fingerprint, target_fingerprint, *weights)
